# Initial kernel scaffold; baseline (speedup 1.0000x reference)
import jax
import jax.numpy as jnp
from jax.experimental import pallas as pl


def kernel(rsa_embeddings, W_timestep, past_kv_pos_offset):
    raise NotImplementedError("write your pallas kernel here")


# speedup vs baseline: 1.6183x; 1.6183x over previous
import functools

import jax
import jax.numpy as jnp
from jax import lax
from jax.experimental import pallas as pl
from jax.experimental.pallas import tpu as pltpu
from jax.experimental.pallas import tpu_sc as plsc

_NC = 2
_NS = 16
_NW = _NC * _NS

_CHUNK = 64


def _sc_gather_broadcast(batch, num_pos, d_model, table_rows):
    p_per_w = num_pos // _NW
    n_chunks = p_per_w // _CHUNK
    mesh = plsc.VectorSubcoreMesh(
        core_axis_name="c", subcore_axis_name="s",
        num_cores=_NC, num_subcores=_NS)

    @functools.partial(
        pl.kernel,
        out_type=jax.ShapeDtypeStruct((batch, num_pos, d_model), jnp.float32),
        mesh=mesh,
        scratch_types=[
            pltpu.VMEM((p_per_w,), jnp.int32),
            pltpu.VMEM((_CHUNK, d_model), jnp.float32),
            pltpu.SemaphoreType.DMA,
        ],
    )
    def k(table_hbm, idx_hbm, out_hbm, idx_v, rows_v, sem):
        wid = lax.axis_index("s") * _NC + lax.axis_index("c")
        base = wid * p_per_w
        pltpu.sync_copy(idx_hbm.at[pl.ds(base, p_per_w)], idx_v)
        for c in range(n_chunks):
            pltpu.async_copy(
                table_hbm.at[idx_v.at[pl.ds(c * _CHUNK, _CHUNK)]],
                rows_v, sem).wait()
            start = base + c * _CHUNK
            for b in range(batch):
                pltpu.sync_copy(rows_v, out_hbm.at[b, pl.ds(start, _CHUNK)])

    return k


def kernel(rsa_embeddings, W_timestep, past_kv_pos_offset):
    batch, num_pos, _ = rsa_embeddings.shape
    table_rows, d_model = W_timestep.shape
    offset = jnp.asarray(past_kv_pos_offset, dtype=jnp.int32)
    idx = (jnp.arange(num_pos, dtype=jnp.int32) + offset) // 3
    k = _sc_gather_broadcast(batch, num_pos, d_model, table_rows)
    return k(W_timestep, idx)
